# baseline (speedup 1.0000x reference)
import functools

import jax
import jax.numpy as jnp
from jax import lax
from jax.experimental import pallas as pl
from jax.experimental.pallas import tpu as pltpu
from jax.experimental.pallas import tpu_sc as plsc

_ALPHA = 0.2
_FAN = 5
_NH = 4
_DH = 64
_DM = _NH * _DH
_NC = 2
_NS = 16
_L = 16


def _gather_body(table_hbm, idx_hbm, out_hbm,
                 idx_v, rows_v, sem, *, b_per_w, chunks):
    wid = lax.axis_index("s") * _NC + lax.axis_index("c")
    base = wid * b_per_w
    pltpu.sync_copy(idx_hbm.at[pl.ds(base, b_per_w)], idx_v)
    descs = []
    off = 0
    for sz in chunks:
        descs.append(pltpu.async_copy(
            table_hbm.at[idx_v.at[pl.ds(off, sz)]],
            rows_v.at[pl.ds(off, sz)], sem))
        off += sz
    for d in descs:
        d.wait()
    pltpu.sync_copy(rows_v, out_hbm.at[pl.ds(base, b_per_w)])


def _sc_gather(table, idx):
    _, d = table.shape
    b = idx.shape[0]
    nw = _NC * _NS
    assert b % (8 * nw) == 0, b
    b_per_w = b // nw
    chunks = []
    r = b_per_w
    while r > 0:
        c = min(128, r)
        chunks.append(c)
        r -= c
    body = functools.partial(_gather_body, b_per_w=b_per_w,
                             chunks=tuple(chunks))
    return pl.kernel(
        body,
        out_type=jax.ShapeDtypeStruct((b, d), table.dtype),
        mesh=plsc.VectorSubcoreMesh(core_axis_name="c", subcore_axis_name="s"),
        scratch_types=[
            pltpu.VMEM((b_per_w,), jnp.int32),
            pltpu.VMEM((b_per_w, d), table.dtype),
            pltpu.SemaphoreType.DMA,
        ],
    )(table, idx)


def _att_stage(gs, gt, cols, al_ref, ar_ref, p_ref):
    c = jnp.dot(gt.astype(jnp.bfloat16), al_ref[...],
                preferred_element_type=jnp.float32)
    es = []
    for k in range(_FAN):
        sk = jnp.dot(gs[k].astype(jnp.bfloat16), ar_ref[...],
                     preferred_element_type=jnp.float32)
        zk = c + sk
        es.append(jnp.where(zk >= 0, zk, _ALPHA * zk))
    m = es[0]
    for k in range(1, _FAN):
        m = jnp.maximum(m, es[k])
    ws = []
    for k in range(_FAN):
        colk = cols[:, k:k + 1]
        mult = jnp.zeros_like(colk, dtype=jnp.float32)
        for l in range(_FAN):
            mult += (cols[:, l:l + 1] == colk).astype(jnp.float32)
        ws.append(jnp.exp(es[k] - m) / mult)
    denom = ws[0]
    for k in range(1, _FAN):
        denom = denom + ws[k]
    inv = 1.0 / denom
    acc = jnp.zeros((gt.shape[0], _DM), jnp.float32)
    for k in range(_FAN):
        attk = ws[k] * inv
        wide = jnp.dot(attk.astype(jnp.bfloat16), p_ref[...],
                       preferred_element_type=jnp.float32)
        gk16 = gs[k].astype(jnp.bfloat16).astype(jnp.float32)
        acc = acc + wide * gk16
    return jnp.where(acc > 0, acc, jnp.exp(acc) - 1.0)


def _fused_body(g0_ref, g1_ref, g2_ref, g3_ref, g4_ref, gt_ref, colsa_ref,
                colsb_ref, w0_ref, al0_ref, ar0_ref, al1_ref, ar1_ref,
                p_ref, w1_ref, wl_ref, bl_ref, o_ref, hb_ref, *,
                n1, n0, bt):
    i = pl.program_id(0)
    nsteps = n1 // bt
    ga = []
    for ref in (g0_ref, g1_ref, g2_ref, g3_ref, g4_ref, gt_ref):
        ga.append(jnp.dot(ref[...], w0_ref[...],
                          preferred_element_type=jnp.float32))
    xa = _att_stage(ga[:_FAN], ga[_FAN], colsa_ref[...],
                    al0_ref, ar0_ref, p_ref)
    hb_ref[pl.ds(i * bt, bt), :] = jnp.dot(
        xa, w1_ref[...],
        preferred_element_type=jnp.float32).astype(jnp.bfloat16)

    @pl.when(i == nsteps - 1)
    def _():
        hb = hb_ref[...]
        colsb = colsb_ref[...]
        iota = lax.broadcasted_iota(jnp.int32, (n0, n1), 1)
        gb = []
        for k in range(_FAN + 1):
            sel = (iota == colsb[:, k:k + 1]).astype(jnp.bfloat16)
            gb.append(jnp.dot(sel, hb, preferred_element_type=jnp.float32))
        xb = _att_stage(gb[:_FAN], gb[_FAN], colsb[:, :_FAN],
                        al1_ref, ar1_ref, p_ref)
        y = jnp.dot(xb, wl_ref[...], preferred_element_type=jnp.float32)
        o_ref[...] = jnp.tanh(y + bl_ref[...])


def _tc_fused(g, colsa, colsb, w0, al0, ar0, al1, ar1, p, w1, wl, blr,
              n1, n0, bt):
    sb = n1 // bt

    def stripe(k):
        return pl.BlockSpec((bt, _DM), lambda i, k=k: (k * sb + i, 0))

    return pl.pallas_call(
        functools.partial(_fused_body, n1=n1, n0=n0, bt=bt),
        grid=(sb,),
        in_specs=[
            stripe(0), stripe(1), stripe(2), stripe(3), stripe(4), stripe(5),
            pl.BlockSpec((bt, _FAN), lambda i: (i, 0)),
            pl.BlockSpec((n0, _FAN + 1), lambda i: (0, 0)),
            pl.BlockSpec((_DM, _DM), lambda i: (0, 0)),
            pl.BlockSpec((_DM, _NH), lambda i: (0, 0)),
            pl.BlockSpec((_DM, _NH), lambda i: (0, 0)),
            pl.BlockSpec((_DM, _NH), lambda i: (0, 0)),
            pl.BlockSpec((_DM, _NH), lambda i: (0, 0)),
            pl.BlockSpec((_NH, _DM), lambda i: (0, 0)),
            pl.BlockSpec((_DM, _DM), lambda i: (0, 0)),
            pl.BlockSpec((_DM, _DM), lambda i: (0, 0)),
            pl.BlockSpec((1, _DM), lambda i: (0, 0)),
        ],
        out_specs=pl.BlockSpec((n0, _DM), lambda i: (0, 0)),
        out_shape=jax.ShapeDtypeStruct((n0, _DM), jnp.float32),
        scratch_shapes=[pltpu.VMEM((n1, _DM), jnp.bfloat16)],
    )(g, g, g, g, g, g, colsa, colsb, w0, al0, ar0, al1, ar1, p, w1,
      wl, blr)


def kernel(feats, sample2_idx, adjA0, adjA1, tioA, adjB0, adjB1, tioB,
           W0, a0, W1, a1, Wl, bl):
    f32 = jnp.float32
    bf16 = jnp.bfloat16
    n1 = adjA1.shape[0] // _FAN
    n0 = adjB1.shape[0] // _FAN

    w0c = W0.transpose(1, 0, 2).reshape(_DM, _DM)
    w1c = W1.transpose(1, 0, 2).reshape(_DM, _DM)
    eye = jnp.eye(_NH, dtype=f32)[:, None, :]
    al0 = (eye * a0[:, :_DH, :]).reshape(_DM, _NH).astype(bf16)
    ar0 = (eye * a0[:, _DH:, :]).reshape(_DM, _NH).astype(bf16)
    al1 = (eye * a1[:, :_DH, :]).reshape(_DM, _NH).astype(bf16)
    ar1 = (eye * a1[:, _DH:, :]).reshape(_DM, _NH).astype(bf16)
    p = (jnp.arange(_DM)[None, :] // _DH
         == jnp.arange(_NH)[:, None]).astype(bf16)
    blr = bl.reshape(1, _DM)

    colsA = adjA1.reshape(n1, _FAN)
    idxA = jnp.concatenate([colsA.T.reshape(-1), adjA0[::_FAN]])
    colsB = adjB1.reshape(n0, _FAN)
    colsB6 = jnp.concatenate([colsB, adjB0[::_FAN, None]], axis=1)

    ga = _sc_gather(feats, sample2_idx[idxA])
    return _tc_fused(ga, colsA, colsB6, w0c, al0, ar0, al1, ar1, p,
                     w1c, Wl, blr, n1, n0, 256)


# speedup vs baseline: 1.1688x; 1.1688x over previous
import functools

import jax
import jax.numpy as jnp
from jax import lax
from jax.experimental import pallas as pl
from jax.experimental.pallas import tpu as pltpu
from jax.experimental.pallas import tpu_sc as plsc

_ALPHA = 0.2
_FAN = 5
_NH = 4
_DH = 64
_DM = _NH * _DH
_NC = 2
_NS = 16
_L = 16


def _gather_body(table_hbm, inner_hbm, idx_hbm, out_hbm,
                 idx_v, cidx_v, rows_v, sem, *, b_per_w, chunks):
    wid = lax.axis_index("s") * _NC + lax.axis_index("c")
    base = wid * b_per_w
    pltpu.sync_copy(idx_hbm.at[pl.ds(base, b_per_w)], idx_v)
    descs = []
    off = 0
    for sz in chunks:
        descs.append(pltpu.async_copy(
            inner_hbm.at[idx_v.at[pl.ds(off, sz)]],
            cidx_v.at[pl.ds(off, sz)], sem))
        off += sz
    for d in descs:
        d.wait()
    descs = []
    off = 0
    for sz in chunks:
        descs.append(pltpu.async_copy(
            table_hbm.at[cidx_v.at[pl.ds(off, sz)]],
            rows_v.at[pl.ds(off, sz)], sem))
        off += sz
    for d in descs:
        d.wait()
    pltpu.sync_copy(rows_v, out_hbm.at[pl.ds(base, b_per_w)])


def _sc_gather(table, inner, idx):
    _, d = table.shape
    b = idx.shape[0]
    nw = _NC * _NS
    assert b % (8 * nw) == 0, b
    b_per_w = b // nw
    chunks = []
    r = b_per_w
    while r > 0:
        c = min(128, r)
        chunks.append(c)
        r -= c
    body = functools.partial(_gather_body, b_per_w=b_per_w,
                             chunks=tuple(chunks))
    return pl.kernel(
        body,
        out_type=jax.ShapeDtypeStruct((b, d), table.dtype),
        mesh=plsc.VectorSubcoreMesh(core_axis_name="c", subcore_axis_name="s"),
        scratch_types=[
            pltpu.VMEM((b_per_w,), jnp.int32),
            pltpu.VMEM((b_per_w,), jnp.int32),
            pltpu.VMEM((b_per_w, d), table.dtype),
            pltpu.SemaphoreType.DMA,
        ],
    )(table, inner, idx)


def _att_stage(gs, gt, cols, al_ref, ar_ref, p_ref):
    c = jnp.dot(gt.astype(jnp.bfloat16), al_ref[...],
                preferred_element_type=jnp.float32)
    es = []
    for k in range(_FAN):
        sk = jnp.dot(gs[k].astype(jnp.bfloat16), ar_ref[...],
                     preferred_element_type=jnp.float32)
        zk = c + sk
        es.append(jnp.where(zk >= 0, zk, _ALPHA * zk))
    m = es[0]
    for k in range(1, _FAN):
        m = jnp.maximum(m, es[k])
    ws = []
    for k in range(_FAN):
        colk = cols[:, k:k + 1]
        mult = jnp.zeros_like(colk, dtype=jnp.float32)
        for l in range(_FAN):
            mult += (cols[:, l:l + 1] == colk).astype(jnp.float32)
        ws.append(jnp.exp(es[k] - m) / mult)
    denom = ws[0]
    for k in range(1, _FAN):
        denom = denom + ws[k]
    inv = 1.0 / denom
    acc = jnp.zeros((gt.shape[0], _DM), jnp.float32)
    for k in range(_FAN):
        attk = ws[k] * inv
        wide = jnp.dot(attk.astype(jnp.bfloat16), p_ref[...],
                       preferred_element_type=jnp.float32)
        gk16 = gs[k].astype(jnp.bfloat16).astype(jnp.float32)
        acc = acc + wide * gk16
    return jnp.where(acc > 0, acc, jnp.exp(acc) - 1.0)


def _fused_body(g0_ref, g1_ref, g2_ref, g3_ref, g4_ref, gt_ref, colsa_ref,
                colsb_ref, w0_ref, al0_ref, ar0_ref, al1_ref, ar1_ref,
                p_ref, w1_ref, wl_ref, bl_ref, o_ref, *, n1, n0):
    ga = []
    for ref in (g0_ref, g1_ref, g2_ref, g3_ref, g4_ref, gt_ref):
        ga.append(jnp.dot(ref[...], w0_ref[...],
                          preferred_element_type=jnp.float32))
    xa = _att_stage(ga[:_FAN], ga[_FAN], colsa_ref[...],
                    al0_ref, ar0_ref, p_ref)
    hb = jnp.dot(xa, w1_ref[...],
                 preferred_element_type=jnp.float32).astype(jnp.bfloat16)
    colsb = colsb_ref[...]
    iota = lax.broadcasted_iota(jnp.int32, (n0, n1), 1)
    gb = []
    for k in range(_FAN + 1):
        sel = (iota == colsb[:, k:k + 1]).astype(jnp.bfloat16)
        gb.append(jnp.dot(sel, hb, preferred_element_type=jnp.float32))
    xb = _att_stage(gb[:_FAN], gb[_FAN], colsb[:, :_FAN],
                    al1_ref, ar1_ref, p_ref)
    y = jnp.dot(xb, wl_ref[...], preferred_element_type=jnp.float32)
    o_ref[...] = jnp.tanh(y + bl_ref[...])


def _tc_fused(g, colsa, colsb, w0, al0, ar0, al1, ar1, p, w1, wl, blr,
              n1, n0):
    def stripe(k):
        return pl.BlockSpec((n1, _DM), lambda i, k=k: (k, 0))

    return pl.pallas_call(
        functools.partial(_fused_body, n1=n1, n0=n0),
        grid=(1,),
        in_specs=[
            stripe(0), stripe(1), stripe(2), stripe(3), stripe(4), stripe(5),
            pl.BlockSpec((n1, _FAN), lambda i: (0, 0)),
            pl.BlockSpec((n0, _FAN + 1), lambda i: (0, 0)),
            pl.BlockSpec((_DM, _DM), lambda i: (0, 0)),
            pl.BlockSpec((_DM, _NH), lambda i: (0, 0)),
            pl.BlockSpec((_DM, _NH), lambda i: (0, 0)),
            pl.BlockSpec((_DM, _NH), lambda i: (0, 0)),
            pl.BlockSpec((_DM, _NH), lambda i: (0, 0)),
            pl.BlockSpec((_NH, _DM), lambda i: (0, 0)),
            pl.BlockSpec((_DM, _DM), lambda i: (0, 0)),
            pl.BlockSpec((_DM, _DM), lambda i: (0, 0)),
            pl.BlockSpec((1, _DM), lambda i: (0, 0)),
        ],
        out_specs=pl.BlockSpec((n0, _DM), lambda i: (0, 0)),
        out_shape=jax.ShapeDtypeStruct((n0, _DM), jnp.float32),
    )(g, g, g, g, g, g, colsa, colsb, w0, al0, ar0, al1, ar1, p, w1,
      wl, blr)


def kernel(feats, sample2_idx, adjA0, adjA1, tioA, adjB0, adjB1, tioB,
           W0, a0, W1, a1, Wl, bl):
    f32 = jnp.float32
    bf16 = jnp.bfloat16
    n1 = adjA1.shape[0] // _FAN
    n0 = adjB1.shape[0] // _FAN

    w0c = W0.transpose(1, 0, 2).reshape(_DM, _DM)
    w1c = W1.transpose(1, 0, 2).reshape(_DM, _DM)
    eye = jnp.eye(_NH, dtype=f32)[:, None, :]
    al0 = (eye * a0[:, :_DH, :]).reshape(_DM, _NH).astype(bf16)
    ar0 = (eye * a0[:, _DH:, :]).reshape(_DM, _NH).astype(bf16)
    al1 = (eye * a1[:, :_DH, :]).reshape(_DM, _NH).astype(bf16)
    ar1 = (eye * a1[:, _DH:, :]).reshape(_DM, _NH).astype(bf16)
    p = (jnp.arange(_DM)[None, :] // _DH
         == jnp.arange(_NH)[:, None]).astype(bf16)
    blr = bl.reshape(1, _DM)

    colsA = adjA1.reshape(n1, _FAN)
    idxA = jnp.concatenate([colsA.T.reshape(-1), adjA0[::_FAN]])
    colsB = adjB1.reshape(n0, _FAN)
    colsB6 = jnp.concatenate([colsB, adjB0[::_FAN, None]], axis=1)

    ga = _sc_gather(feats, sample2_idx, idxA)
    return _tc_fused(ga, colsA, colsB6, w0c, al0, ar0, al1, ar1, p,
                     w1c, Wl, blr, n1, n0)


# speedup vs baseline: 1.1710x; 1.0019x over previous
import functools

import jax
import jax.numpy as jnp
from jax import lax
from jax.experimental import pallas as pl
from jax.experimental.pallas import tpu as pltpu
from jax.experimental.pallas import tpu_sc as plsc

_ALPHA = 0.2
_FAN = 5
_NH = 4
_DH = 64
_DM = _NH * _DH
_NC = 2
_NS = 16
_L = 16


def _gather_body(table_hbm, inner_hbm, idx_hbm, out_hbm,
                 idx_v, cidx_v, rows_v, sem, *, b_per_w, chunks):
    wid = lax.axis_index("s") * _NC + lax.axis_index("c")
    base = wid * b_per_w
    pltpu.sync_copy(idx_hbm.at[pl.ds(base, b_per_w)], idx_v)
    descs = []
    off = 0
    for sz in chunks:
        descs.append(pltpu.async_copy(
            inner_hbm.at[idx_v.at[pl.ds(off, sz)]],
            cidx_v.at[pl.ds(off, sz)], sem))
        off += sz
    for d in descs:
        d.wait()
    descs = []
    off = 0
    for sz in chunks:
        descs.append(pltpu.async_copy(
            table_hbm.at[cidx_v.at[pl.ds(off, sz)]],
            rows_v.at[pl.ds(off, sz)], sem))
        off += sz
    for d in descs:
        d.wait()
    pltpu.sync_copy(rows_v, out_hbm.at[pl.ds(base, b_per_w)])


def _sc_gather(table, inner, idx):
    _, d = table.shape
    b = idx.shape[0]
    nw = _NC * _NS
    assert b % (8 * nw) == 0, b
    b_per_w = b // nw
    chunks = []
    r = b_per_w
    while r > 0:
        c = min(128, r)
        chunks.append(c)
        r -= c
    body = functools.partial(_gather_body, b_per_w=b_per_w,
                             chunks=tuple(chunks))
    return pl.kernel(
        body,
        out_type=jax.ShapeDtypeStruct((b, d), table.dtype),
        mesh=plsc.VectorSubcoreMesh(core_axis_name="c", subcore_axis_name="s"),
        scratch_types=[
            pltpu.VMEM((b_per_w,), jnp.int32),
            pltpu.VMEM((b_per_w,), jnp.int32),
            pltpu.VMEM((b_per_w, d), table.dtype),
            pltpu.SemaphoreType.DMA,
        ],
    )(table, inner, idx)


def _att_stage(gs, gt, cols, al_ref, ar_ref, p_ref):
    c = jnp.dot(gt.astype(jnp.bfloat16), al_ref[...],
                preferred_element_type=jnp.float32)
    es = []
    for k in range(_FAN):
        sk = jnp.dot(gs[k].astype(jnp.bfloat16), ar_ref[...],
                     preferred_element_type=jnp.float32)
        zk = c + sk
        es.append(jnp.where(zk >= 0, zk, _ALPHA * zk))
    m = es[0]
    for k in range(1, _FAN):
        m = jnp.maximum(m, es[k])
    ws = []
    for k in range(_FAN):
        colk = cols[:, k:k + 1]
        mult = jnp.zeros_like(colk, dtype=jnp.float32)
        for l in range(_FAN):
            mult += (cols[:, l:l + 1] == colk).astype(jnp.float32)
        ws.append(jnp.exp(es[k] - m) / mult)
    denom = ws[0]
    for k in range(1, _FAN):
        denom = denom + ws[k]
    inv = 1.0 / denom
    acc = jnp.zeros((gt.shape[0], _DM), jnp.float32)
    for k in range(_FAN):
        attk = ws[k] * inv
        wide = jnp.dot(attk.astype(jnp.bfloat16), p_ref[...],
                       preferred_element_type=jnp.float32)
        gk16 = gs[k].astype(jnp.bfloat16).astype(jnp.float32)
        acc = acc + wide * gk16
    return jnp.where(acc > 0, acc, jnp.exp(acc) - 1.0)


def _fused_body(g0_ref, g1_ref, g2_ref, g3_ref, g4_ref, gt_ref, colsa_ref,
                colsb_ref, w0_ref, al0_ref, ar0_ref, al1_ref, ar1_ref,
                p_ref, w1_ref, wl_ref, bl_ref, o_ref, hb_ref, *,
                n1, n0, bt):
    i = pl.program_id(0)
    nsteps = n1 // bt
    ga = []
    for ref in (g0_ref, g1_ref, g2_ref, g3_ref, g4_ref, gt_ref):
        ga.append(jnp.dot(ref[...], w0_ref[...],
                          preferred_element_type=jnp.float32))
    xa = _att_stage(ga[:_FAN], ga[_FAN], colsa_ref[...],
                    al0_ref, ar0_ref, p_ref)
    hb_ref[pl.ds(i * bt, bt), :] = jnp.dot(
        xa, w1_ref[...],
        preferred_element_type=jnp.float32).astype(jnp.bfloat16)

    @pl.when(i == nsteps - 1)
    def _():
        hb = hb_ref[...]
        colsb = colsb_ref[...]
        iota = lax.broadcasted_iota(jnp.int32, (n0, n1), 1)
        gb = []
        for k in range(_FAN + 1):
            sel = (iota == colsb[:, k:k + 1]).astype(jnp.bfloat16)
            gb.append(jnp.dot(sel, hb, preferred_element_type=jnp.float32))
        xb = _att_stage(gb[:_FAN], gb[_FAN], colsb[:, :_FAN],
                        al1_ref, ar1_ref, p_ref)
        y = jnp.dot(xb, wl_ref[...], preferred_element_type=jnp.float32)
        o_ref[...] = jnp.tanh(y + bl_ref[...])


def _tc_fused(g, colsa, colsb, w0, al0, ar0, al1, ar1, p, w1, wl, blr,
              n1, n0, bt=768):
    sb = n1 // bt

    def stripe(k):
        return pl.BlockSpec((bt, _DM), lambda i, k=k: (k * sb + i, 0))

    return pl.pallas_call(
        functools.partial(_fused_body, n1=n1, n0=n0, bt=bt),
        grid=(sb,),
        in_specs=[
            stripe(0), stripe(1), stripe(2), stripe(3), stripe(4), stripe(5),
            pl.BlockSpec((bt, _FAN), lambda i: (i, 0)),
            pl.BlockSpec((n0, _FAN + 1), lambda i: (0, 0)),
            pl.BlockSpec((_DM, _DM), lambda i: (0, 0)),
            pl.BlockSpec((_DM, _NH), lambda i: (0, 0)),
            pl.BlockSpec((_DM, _NH), lambda i: (0, 0)),
            pl.BlockSpec((_DM, _NH), lambda i: (0, 0)),
            pl.BlockSpec((_DM, _NH), lambda i: (0, 0)),
            pl.BlockSpec((_NH, _DM), lambda i: (0, 0)),
            pl.BlockSpec((_DM, _DM), lambda i: (0, 0)),
            pl.BlockSpec((_DM, _DM), lambda i: (0, 0)),
            pl.BlockSpec((1, _DM), lambda i: (0, 0)),
        ],
        out_specs=pl.BlockSpec((n0, _DM), lambda i: (0, 0)),
        out_shape=jax.ShapeDtypeStruct((n0, _DM), jnp.float32),
        scratch_shapes=[pltpu.VMEM((n1, _DM), jnp.bfloat16)],
    )(g, g, g, g, g, g, colsa, colsb, w0, al0, ar0, al1, ar1, p, w1,
      wl, blr)


def kernel(feats, sample2_idx, adjA0, adjA1, tioA, adjB0, adjB1, tioB,
           W0, a0, W1, a1, Wl, bl):
    f32 = jnp.float32
    bf16 = jnp.bfloat16
    n1 = adjA1.shape[0] // _FAN
    n0 = adjB1.shape[0] // _FAN

    w0c = W0.transpose(1, 0, 2).reshape(_DM, _DM)
    w1c = W1.transpose(1, 0, 2).reshape(_DM, _DM)
    eye = jnp.eye(_NH, dtype=f32)[:, None, :]
    al0 = (eye * a0[:, :_DH, :]).reshape(_DM, _NH).astype(bf16)
    ar0 = (eye * a0[:, _DH:, :]).reshape(_DM, _NH).astype(bf16)
    al1 = (eye * a1[:, :_DH, :]).reshape(_DM, _NH).astype(bf16)
    ar1 = (eye * a1[:, _DH:, :]).reshape(_DM, _NH).astype(bf16)
    p = (jnp.arange(_DM)[None, :] // _DH
         == jnp.arange(_NH)[:, None]).astype(bf16)
    blr = bl.reshape(1, _DM)

    colsA = adjA1.reshape(n1, _FAN)
    idxA = jnp.concatenate([colsA.T.reshape(-1), adjA0[::_FAN]])
    colsB = adjB1.reshape(n0, _FAN)
    colsB6 = jnp.concatenate([colsB, adjB0[::_FAN, None]], axis=1)

    ga = _sc_gather(feats, sample2_idx, idxA)
    return _tc_fused(ga, colsA, colsB6, w0c, al0, ar0, al1, ar1, p,
                     w1c, Wl, blr, n1, n0)


# speedup vs baseline: 1.1743x; 1.0028x over previous
import functools

import jax
import jax.numpy as jnp
from jax import lax
from jax.experimental import pallas as pl
from jax.experimental.pallas import tpu as pltpu
from jax.experimental.pallas import tpu_sc as plsc

_ALPHA = 0.2
_FAN = 5
_NH = 4
_DH = 64
_DM = _NH * _DH
_NC = 2
_NS = 16
_L = 16


def _gather_body(table_hbm, inner_hbm, idx_hbm, out_hbm,
                 idx_v, cidx_v, rows_v, csem, sem, *, b_per_w, chunks):
    wid = lax.axis_index("s") * _NC + lax.axis_index("c")
    base = wid * b_per_w
    pltpu.sync_copy(idx_hbm.at[pl.ds(base, b_per_w)], idx_v)
    cdescs = []
    off = 0
    for sz in chunks:
        cdescs.append(pltpu.async_copy(
            inner_hbm.at[idx_v.at[pl.ds(off, sz)]],
            cidx_v.at[pl.ds(off, sz)], csem))
        off += sz
    rdescs = []
    off = 0
    for c, sz in enumerate(chunks):
        cdescs[c].wait()
        rdescs.append(pltpu.async_copy(
            table_hbm.at[cidx_v.at[pl.ds(off, sz)]],
            rows_v.at[pl.ds(off, sz)], sem))
        off += sz
    for d in rdescs:
        d.wait()
    pltpu.sync_copy(rows_v, out_hbm.at[pl.ds(base, b_per_w)])


def _sc_gather(table, inner, idx):
    _, d = table.shape
    b = idx.shape[0]
    nw = _NC * _NS
    assert b % (8 * nw) == 0, b
    b_per_w = b // nw
    chunks = []
    r = b_per_w
    while r > 0:
        c = min(128, r)
        chunks.append(c)
        r -= c
    body = functools.partial(_gather_body, b_per_w=b_per_w,
                             chunks=tuple(chunks))
    return pl.kernel(
        body,
        out_type=jax.ShapeDtypeStruct((b, d), table.dtype),
        mesh=plsc.VectorSubcoreMesh(core_axis_name="c", subcore_axis_name="s"),
        scratch_types=[
            pltpu.VMEM((b_per_w,), jnp.int32),
            pltpu.VMEM((b_per_w,), jnp.int32),
            pltpu.VMEM((b_per_w, d), table.dtype),
            pltpu.SemaphoreType.DMA,
            pltpu.SemaphoreType.DMA,
        ],
    )(table, inner, idx)


def _att_stage(gs, gt, cols, al_ref, ar_ref, p_ref):
    c = jnp.dot(gt.astype(jnp.bfloat16), al_ref[...],
                preferred_element_type=jnp.float32)
    es = []
    for k in range(_FAN):
        sk = jnp.dot(gs[k].astype(jnp.bfloat16), ar_ref[...],
                     preferred_element_type=jnp.float32)
        zk = c + sk
        es.append(jnp.where(zk >= 0, zk, _ALPHA * zk))
    m = es[0]
    for k in range(1, _FAN):
        m = jnp.maximum(m, es[k])
    ws = []
    for k in range(_FAN):
        colk = cols[:, k:k + 1]
        mult = jnp.zeros_like(colk, dtype=jnp.float32)
        for l in range(_FAN):
            mult += (cols[:, l:l + 1] == colk).astype(jnp.float32)
        ws.append(jnp.exp(es[k] - m) / mult)
    denom = ws[0]
    for k in range(1, _FAN):
        denom = denom + ws[k]
    inv = 1.0 / denom
    acc = jnp.zeros((gt.shape[0], _DM), jnp.float32)
    for k in range(_FAN):
        attk = ws[k] * inv
        wide = jnp.dot(attk.astype(jnp.bfloat16), p_ref[...],
                       preferred_element_type=jnp.float32)
        gk16 = gs[k].astype(jnp.bfloat16).astype(jnp.float32)
        acc = acc + wide * gk16
    return jnp.where(acc > 0, acc, jnp.exp(acc) - 1.0)


def _fused_body(g0_ref, g1_ref, g2_ref, g3_ref, g4_ref, gt_ref, colsa_ref,
                colsb_ref, w0_ref, al0_ref, ar0_ref, al1_ref, ar1_ref,
                p_ref, w1_ref, wl_ref, bl_ref, o_ref, hb_ref, *,
                n1, n0, bt):
    i = pl.program_id(0)
    nsteps = n1 // bt
    ga = []
    for ref in (g0_ref, g1_ref, g2_ref, g3_ref, g4_ref, gt_ref):
        ga.append(jnp.dot(ref[...], w0_ref[...],
                          preferred_element_type=jnp.float32))
    xa = _att_stage(ga[:_FAN], ga[_FAN], colsa_ref[...],
                    al0_ref, ar0_ref, p_ref)
    hb_ref[pl.ds(i * bt, bt), :] = jnp.dot(
        xa, w1_ref[...],
        preferred_element_type=jnp.float32).astype(jnp.bfloat16)

    @pl.when(i == nsteps - 1)
    def _():
        hb = hb_ref[...]
        colsb = colsb_ref[...]
        iota = lax.broadcasted_iota(jnp.int32, (n0, n1), 1)
        gb = []
        for k in range(_FAN + 1):
            sel = (iota == colsb[:, k:k + 1]).astype(jnp.bfloat16)
            gb.append(jnp.dot(sel, hb, preferred_element_type=jnp.float32))
        xb = _att_stage(gb[:_FAN], gb[_FAN], colsb[:, :_FAN],
                        al1_ref, ar1_ref, p_ref)
        y = jnp.dot(xb, wl_ref[...], preferred_element_type=jnp.float32)
        o_ref[...] = jnp.tanh(y + bl_ref[...])


def _tc_fused(g, colsa, colsb, w0, al0, ar0, al1, ar1, p, w1, wl, blr,
              n1, n0, bt=768):
    sb = n1 // bt

    def stripe(k):
        return pl.BlockSpec((bt, _DM), lambda i, k=k: (k * sb + i, 0))

    return pl.pallas_call(
        functools.partial(_fused_body, n1=n1, n0=n0, bt=bt),
        grid=(sb,),
        in_specs=[
            stripe(0), stripe(1), stripe(2), stripe(3), stripe(4), stripe(5),
            pl.BlockSpec((bt, _FAN), lambda i: (i, 0)),
            pl.BlockSpec((n0, _FAN + 1), lambda i: (0, 0)),
            pl.BlockSpec((_DM, _DM), lambda i: (0, 0)),
            pl.BlockSpec((_DM, _NH), lambda i: (0, 0)),
            pl.BlockSpec((_DM, _NH), lambda i: (0, 0)),
            pl.BlockSpec((_DM, _NH), lambda i: (0, 0)),
            pl.BlockSpec((_DM, _NH), lambda i: (0, 0)),
            pl.BlockSpec((_NH, _DM), lambda i: (0, 0)),
            pl.BlockSpec((_DM, _DM), lambda i: (0, 0)),
            pl.BlockSpec((_DM, _DM), lambda i: (0, 0)),
            pl.BlockSpec((1, _DM), lambda i: (0, 0)),
        ],
        out_specs=pl.BlockSpec((n0, _DM), lambda i: (0, 0)),
        out_shape=jax.ShapeDtypeStruct((n0, _DM), jnp.float32),
        scratch_shapes=[pltpu.VMEM((n1, _DM), jnp.bfloat16)],
    )(g, g, g, g, g, g, colsa, colsb, w0, al0, ar0, al1, ar1, p, w1,
      wl, blr)


def kernel(feats, sample2_idx, adjA0, adjA1, tioA, adjB0, adjB1, tioB,
           W0, a0, W1, a1, Wl, bl):
    f32 = jnp.float32
    bf16 = jnp.bfloat16
    n1 = adjA1.shape[0] // _FAN
    n0 = adjB1.shape[0] // _FAN

    w0c = W0.transpose(1, 0, 2).reshape(_DM, _DM)
    w1c = W1.transpose(1, 0, 2).reshape(_DM, _DM)
    eye = jnp.eye(_NH, dtype=f32)[:, None, :]
    al0 = (eye * a0[:, :_DH, :]).reshape(_DM, _NH).astype(bf16)
    ar0 = (eye * a0[:, _DH:, :]).reshape(_DM, _NH).astype(bf16)
    al1 = (eye * a1[:, :_DH, :]).reshape(_DM, _NH).astype(bf16)
    ar1 = (eye * a1[:, _DH:, :]).reshape(_DM, _NH).astype(bf16)
    p = (jnp.arange(_DM)[None, :] // _DH
         == jnp.arange(_NH)[:, None]).astype(bf16)
    blr = bl.reshape(1, _DM)

    colsA = adjA1.reshape(n1, _FAN)
    idxA = jnp.concatenate([colsA.T.reshape(-1), adjA0[::_FAN]])
    colsB = adjB1.reshape(n0, _FAN)
    colsB6 = jnp.concatenate([colsB, adjB0[::_FAN, None]], axis=1)

    ga = _sc_gather(feats, sample2_idx, idxA)
    return _tc_fused(ga, colsA, colsB6, w0c, al0, ar0, al1, ar1, p,
                     w1c, Wl, blr, n1, n0)
